# baseline (speedup 1.0000x reference)
import functools

import jax
import jax.numpy as jnp
from jax import lax
from jax.experimental import pallas as pl
from jax.experimental.pallas import tpu as pltpu
from jax.experimental.pallas import tpu_sc as plsc

N = 10000
E = 320000
D = 128
HID = 128
OUT = 32
ALPHA = 0.01

NC, NS = 2, 16
NW = NC * NS
EW = E // NW
CH = 80
NCHUNK = EW // CH
RPS = 624
NB = 4

_mesh = plsc.VectorSubcoreMesh(
    core_axis_name="c", subcore_axis_name="s", num_cores=NC, num_subcores=NS)

_sc_params = pltpu.CompilerParams(
    use_tc_tiling_on_sc=False, needs_layout_passes=False)


def _zero_fill(zbuf):
    z = jnp.zeros((16,), jnp.float32)

    def zb(j, carry):
        zbuf[j, pl.ds(0, 16)] = z
        zbuf[j, pl.ds(16, 16)] = z
        return carry

    lax.fori_loop(0, RPS, zb, 0)


def _zero_deg(deg_v):
    z = jnp.zeros((16,), jnp.float32)

    def zb(j, carry):
        deg_v[j, pl.ds(0, 16)] = z
        return carry

    lax.fori_loop(0, N // 16, zb, 0)


def _zero_acc(sid, zbuf, acc):
    pltpu.sync_copy(zbuf, acc.at[pl.ds(sid * RPS, RPS)])

    @pl.when(sid == NS - 1)
    def _():
        pltpu.sync_copy(zbuf.at[pl.ds(0, 16)], acc.at[pl.ds(NS * RPS, 16)])


def _write_acc(cid, sid, acc, out, sem):
    copies = [pltpu.make_async_copy(acc.at[pl.ds(sid * RPS, RPS)],
                                    out.at[cid, pl.ds(sid * RPS, RPS)], sem)]
    tail = pltpu.make_async_copy(acc.at[pl.ds(NS * RPS, 16)],
                                 out.at[cid, pl.ds(NS * RPS, 16)], sem)
    copies[0].start()

    @pl.when(sid == NS - 1)
    def _():
        tail.start()

    return copies[0], tail


def _wait_acc(sid, main, tail):
    main.wait()

    @pl.when(sid == NS - 1)
    def _():
        tail.wait()


def _edge_pass(table, idx_s, idx_d, acc, rows, gsems, ssems, deg_v):
    ones = jnp.ones((16,), jnp.float32)

    def hist(j):
        if deg_v is None:
            return
        for k in range(CH // 16):
            v = idx_d[j, pl.ds(k * 16, 16)]
            plsc.addupdate_scatter(
                deg_v, [lax.shift_right_logical(v, 4),
                        lax.bitwise_and(v, 15)], ones)

    def step(j, b):
        bn = (b + NB - 1) % NB
        pltpu.make_async_copy(table.at[idx_s.at[j]], rows[b], gsems[b]).wait()
        pltpu.async_copy(rows[b], acc.at[idx_d.at[j]], ssems[b], add=True)
        hist(j)

        @pl.when(j + NB - 1 < NCHUNK)
        def _():
            @pl.when(j >= 1)
            def _():
                pltpu.make_async_copy(
                    rows[bn], acc.at[idx_d.at[j - 1]], ssems[bn]).wait()
            pltpu.async_copy(table.at[idx_s.at[j + NB - 1]], rows[bn],
                             gsems[bn])

    def eb(j, carry):
        for b in range(NB):
            @pl.when(j % NB == b)
            def _(b=b):
                step(j, b)
        return carry

    for k in range(NB - 1):
        pltpu.async_copy(table.at[idx_s.at[k]], rows[k], gsems[k])
    lax.fori_loop(0, NCHUNK, eb, 0)
    for j in range(NCHUNK - NB, NCHUNK):
        b = j % NB
        pltpu.make_async_copy(rows[b], acc.at[idx_d.at[j]], ssems[b]).wait()


@functools.partial(
    pl.kernel,
    out_type=(
        jax.ShapeDtypeStruct((NC, N, OUT), jnp.float32),
        jax.ShapeDtypeStruct((NC, N, OUT), jnp.float32),
        jax.ShapeDtypeStruct((NW, N // 16, 16), jnp.float32),
        jax.ShapeDtypeStruct((NW, N // 16, 16), jnp.float32),
    ),
    mesh=_mesh,
    compiler_params=_sc_params,
    scratch_types=[
        pltpu.VMEM((NCHUNK, CH), jnp.int32),
        pltpu.VMEM((NCHUNK, CH), jnp.int32),
        [pltpu.VMEM((CH, OUT), jnp.float32)] * NB,
        pltpu.VMEM((RPS, OUT), jnp.float32),
        pltpu.VMEM((N // 16, 16), jnp.float32),
        pltpu.VMEM_SHARED((N, OUT), jnp.float32),
        pltpu.VMEM_SHARED((N, OUT), jnp.float32),
        [pltpu.SemaphoreType.DMA] * NB,
        [pltpu.SemaphoreType.DMA] * NB,
        pltpu.SemaphoreType.DMA,
    ],
)
def _sc_layer1(g_m, g_u, src_mu, dst_mu, src_um, dst_um,
               su_out, s1_out, dum_out, dmu_out,
               idx_s, idx_d, rows, zbuf, deg_v, acc_u, acc_m,
               gsems, ssems, wsem):
    cid = lax.axis_index("c")
    sid = lax.axis_index("s")
    wid = cid * NS + sid

    _zero_fill(zbuf)
    _zero_acc(sid, zbuf, acc_u)
    _zero_acc(sid, zbuf, acc_m)
    plsc.subcore_barrier()

    pltpu.sync_copy(src_mu.at[wid], idx_s)
    pltpu.sync_copy(dst_mu.at[wid], idx_d)
    _zero_deg(deg_v)
    _edge_pass(g_m, idx_s, idx_d, acc_u, rows, gsems, ssems, deg_v)
    pltpu.sync_copy(deg_v, dmu_out.at[wid])
    plsc.subcore_barrier()
    w_main, w_tail = _write_acc(cid, sid, acc_u, su_out, wsem)

    pltpu.sync_copy(src_um.at[wid], idx_s)
    pltpu.sync_copy(dst_um.at[wid], idx_d)
    _zero_deg(deg_v)
    _edge_pass(g_u, idx_s, idx_d, acc_m, rows, gsems, ssems, deg_v)
    pltpu.sync_copy(deg_v, dum_out.at[wid])
    plsc.subcore_barrier()
    pltpu.sync_copy(acc_m.at[pl.ds(sid * RPS, RPS)],
                    s1_out.at[cid, pl.ds(sid * RPS, RPS)])

    @pl.when(sid == NS - 1)
    def _():
        pltpu.sync_copy(acc_m.at[pl.ds(NS * RPS, 16)],
                        s1_out.at[cid, pl.ds(NS * RPS, 16)])

    _wait_acc(sid, w_main, w_tail)


@functools.partial(
    pl.kernel,
    out_type=jax.ShapeDtypeStruct((NC, N, OUT), jnp.float32),
    mesh=_mesh,
    compiler_params=_sc_params,
    scratch_types=[
        pltpu.VMEM((NCHUNK, CH), jnp.int32),
        pltpu.VMEM((NCHUNK, CH), jnp.int32),
        [pltpu.VMEM((CH, OUT), jnp.float32)] * NB,
        pltpu.VMEM((RPS, OUT), jnp.float32),
        pltpu.VMEM_SHARED((N, OUT), jnp.float32),
        [pltpu.SemaphoreType.DMA] * NB,
        [pltpu.SemaphoreType.DMA] * NB,
    ],
)
def _sc_segsum(table, src, dst, out, idx_s, idx_d, rows, zbuf, acc,
               gsems, ssems):
    cid = lax.axis_index("c")
    sid = lax.axis_index("s")
    wid = cid * NS + sid

    _zero_fill(zbuf)
    _zero_acc(sid, zbuf, acc)
    plsc.subcore_barrier()

    pltpu.sync_copy(src.at[wid], idx_s)
    pltpu.sync_copy(dst.at[wid], idx_d)
    _edge_pass(table, idx_s, idx_d, acc, rows, gsems, ssems, None)

    plsc.subcore_barrier()
    pltpu.sync_copy(acc.at[pl.ds(sid * RPS, RPS)],
                    out.at[cid, pl.ds(sid * RPS, RPS)])

    @pl.when(sid == NS - 1)
    def _():
        pltpu.sync_copy(acc.at[pl.ds(NS * RPS, 16)],
                        out.at[cid, pl.ds(NS * RPS, 16)])


BM = 1000


def _dense_body(x_ref, w1_ref, b1_ref, w2_ref, o_ref):
    h = jnp.dot(x_ref[...], w1_ref[0], preferred_element_type=jnp.float32)
    h = jnp.maximum(h + b1_ref[0], 0.0)
    o_ref[...] = jnp.dot(h, w2_ref[...], preferred_element_type=jnp.float32)


def _dense_proj(X, W1, B1, W2):
    nb = X.shape[0] // BM
    per = nb // 2
    return pl.pallas_call(
        _dense_body,
        grid=(nb,),
        in_specs=[
            pl.BlockSpec((BM, D), lambda i: (i, 0)),
            pl.BlockSpec((1, D, HID), lambda i: (i // per, 0, 0)),
            pl.BlockSpec((1, 1, HID), lambda i: (i // per, 0, 0)),
            pl.BlockSpec((HID, OUT), lambda i: (0, 0)),
        ],
        out_specs=pl.BlockSpec((BM, OUT), lambda i: (i, 0)),
        out_shape=jax.ShapeDtypeStruct((X.shape[0], OUT), jnp.float32),
    )(X, W1, B1, W2)


def _prep_body(su_ref, dmu_ref, dum_ref, t_ref, dm_ref):
    du = jnp.maximum(jnp.sum(dmu_ref[...], axis=0), 1.0)
    t_ref[...] = (su_ref[0] + su_ref[1]) / du[:, None]
    dm_ref[...] = jnp.maximum(jnp.sum(dum_ref[...], axis=0), 1.0)[:, None]


def _prep(su, dmu, dum):
    return pl.pallas_call(
        _prep_body,
        grid=(1,),
        in_specs=[
            pl.BlockSpec((NC, N, OUT), lambda i: (0, 0, 0)),
            pl.BlockSpec((NW, N), lambda i: (0, 0)),
            pl.BlockSpec((NW, N), lambda i: (0, 0)),
        ],
        out_specs=[
            pl.BlockSpec((N, OUT), lambda i: (0, 0)),
            pl.BlockSpec((N, 1), lambda i: (0, 0)),
        ],
        out_shape=[
            jax.ShapeDtypeStruct((N, OUT), jnp.float32),
            jax.ShapeDtypeStruct((N, 1), jnp.float32),
        ],
    )(su, dmu, dum)


def _final_body(g_ref, s1_ref, s2_ref, d_ref, b_ref, o_ref):
    num = (2.0 * ALPHA) * (s1_ref[0] + s1_ref[1]) + (s2_ref[0] + s2_ref[1])
    o_ref[...] = (ALPHA * ALPHA) * g_ref[...] + num / d_ref[...] + b_ref[...]


def _final(g_m, s1, s2, deg_m, b_out):
    return pl.pallas_call(
        _final_body,
        grid=(N // BM,),
        in_specs=[
            pl.BlockSpec((BM, OUT), lambda i: (i, 0)),
            pl.BlockSpec((2, BM, OUT), lambda i: (0, i, 0)),
            pl.BlockSpec((2, BM, OUT), lambda i: (0, i, 0)),
            pl.BlockSpec((BM, 1), lambda i: (i, 0)),
            pl.BlockSpec((1, OUT), lambda i: (0, 0)),
        ],
        out_specs=pl.BlockSpec((BM, OUT), lambda i: (i, 0)),
        out_shape=jax.ShapeDtypeStruct((N, OUT), jnp.float32),
    )(g_m, s1, s2, deg_m, b_out)


def kernel(x_movie, x_user, edge_index_um, edge_index_mu, W_in_movie,
           b_in_movie, W_in_user, b_in_user, W_out, b_out):
    X = jnp.concatenate([x_movie, x_user], axis=0)
    W1 = jnp.stack([W_in_movie, W_in_user])
    B1 = jnp.stack([b_in_movie, b_in_user]).reshape(2, 1, HID)
    G = _dense_proj(X, W1, B1, W_out)
    g_m, g_u = G[:N], G[N:]

    src_um = edge_index_um[0].astype(jnp.int32).reshape(NW, NCHUNK, CH)
    dst_um = edge_index_um[1].astype(jnp.int32).reshape(NW, NCHUNK, CH)
    src_mu = edge_index_mu[0].astype(jnp.int32).reshape(NW, NCHUNK, CH)
    dst_mu = edge_index_mu[1].astype(jnp.int32).reshape(NW, NCHUNK, CH)

    su, s1, dum, dmu = _sc_layer1(g_m, g_u, src_mu, dst_mu, src_um, dst_um)
    t_u, deg_m = _prep(su, dmu.reshape(NW, N), dum.reshape(NW, N))
    s2 = _sc_segsum(t_u, src_um, dst_um)
    return _final(g_m, s1, s2, deg_m, b_out.reshape(1, OUT))


# speedup vs baseline: 1.0937x; 1.0937x over previous
import functools

import jax
import jax.numpy as jnp
from jax import lax
from jax.experimental import pallas as pl
from jax.experimental.pallas import tpu as pltpu
from jax.experimental.pallas import tpu_sc as plsc

N = 10000
E = 320000
D = 128
HID = 128
OUT = 32
ALPHA = 0.01

NC, NS = 2, 16
NW = NC * NS
EW = E // NW
CH = 80
NCHUNK = EW // CH
RPS = 624
NB = 4

_mesh = plsc.VectorSubcoreMesh(
    core_axis_name="c", subcore_axis_name="s", num_cores=NC, num_subcores=NS)

_sc_params = pltpu.CompilerParams(
    use_tc_tiling_on_sc=False, needs_layout_passes=False)


def _zero_fill(zbuf):
    z = jnp.zeros((16,), jnp.float32)

    def zb(j, carry):
        zbuf[j, pl.ds(0, 16)] = z
        zbuf[j, pl.ds(16, 16)] = z
        return carry

    lax.fori_loop(0, RPS, zb, 0)


DROWS = 640


def _zero_deg(deg_v):
    z = jnp.zeros((16,), jnp.float32)

    def zb(j, carry):
        deg_v[j, pl.ds(0, 16)] = z
        return carry

    lax.fori_loop(0, DROWS, zb, 0)


def _build_identity_idx(idx_id):
    base = lax.iota(jnp.int32, 16)
    for r in range(5):
        for m in range(8):
            idx_id[r, pl.ds(16 * m, 16)] = base + (128 * r + 16 * m)


def _reduce_deg(deg_v, idx_id, deg_sh):
    for r in range(5):
        pltpu.sync_copy(deg_v.at[pl.ds(128 * r, 128)],
                        deg_sh.at[idx_id.at[r]], add=True)


def _zero_acc(sid, zbuf, acc):
    pltpu.sync_copy(zbuf, acc.at[pl.ds(sid * RPS, RPS)])

    @pl.when(sid == NS - 1)
    def _():
        pltpu.sync_copy(zbuf.at[pl.ds(0, 16)], acc.at[pl.ds(NS * RPS, 16)])


def _write_acc(cid, sid, acc, out, sem):
    copies = [pltpu.make_async_copy(acc.at[pl.ds(sid * RPS, RPS)],
                                    out.at[cid, pl.ds(sid * RPS, RPS)], sem)]
    tail = pltpu.make_async_copy(acc.at[pl.ds(NS * RPS, 16)],
                                 out.at[cid, pl.ds(NS * RPS, 16)], sem)
    copies[0].start()

    @pl.when(sid == NS - 1)
    def _():
        tail.start()

    return copies[0], tail


def _wait_acc(sid, main, tail):
    main.wait()

    @pl.when(sid == NS - 1)
    def _():
        tail.wait()


def _edge_pass(table, idx_s, idx_d, acc, rows, gsems, ssems, deg_v):
    ones = jnp.ones((16,), jnp.float32)

    def hist(j):
        if deg_v is None:
            return
        for k in range(CH // 16):
            v = idx_d[j, pl.ds(k * 16, 16)]
            plsc.addupdate_scatter(
                deg_v, [lax.shift_right_logical(v, 4),
                        lax.bitwise_and(v, 15)], ones)

    def step(j, b):
        bn = (b + NB - 1) % NB
        pltpu.make_async_copy(table.at[idx_s.at[j]], rows[b], gsems[b]).wait()
        pltpu.async_copy(rows[b], acc.at[idx_d.at[j]], ssems[b], add=True)
        hist(j)

        @pl.when(j + NB - 1 < NCHUNK)
        def _():
            @pl.when(j >= 1)
            def _():
                pltpu.make_async_copy(
                    rows[bn], acc.at[idx_d.at[j - 1]], ssems[bn]).wait()
            pltpu.async_copy(table.at[idx_s.at[j + NB - 1]], rows[bn],
                             gsems[bn])

    def eb(j, carry):
        for b in range(NB):
            @pl.when(j % NB == b)
            def _(b=b):
                step(j, b)
        return carry

    for k in range(NB - 1):
        pltpu.async_copy(table.at[idx_s.at[k]], rows[k], gsems[k])
    lax.fori_loop(0, NCHUNK, eb, 0)
    for j in range(NCHUNK - NB, NCHUNK):
        b = j % NB
        pltpu.make_async_copy(rows[b], acc.at[idx_d.at[j]], ssems[b]).wait()


@functools.partial(
    pl.kernel,
    out_type=(
        jax.ShapeDtypeStruct((NC, N, OUT), jnp.float32),
        jax.ShapeDtypeStruct((NC, N, OUT), jnp.float32),
        jax.ShapeDtypeStruct((NC, DROWS, 16), jnp.float32),
        jax.ShapeDtypeStruct((NC, DROWS, 16), jnp.float32),
    ),
    mesh=_mesh,
    compiler_params=_sc_params,
    scratch_types=[
        pltpu.VMEM((NCHUNK, CH), jnp.int32),
        pltpu.VMEM((NCHUNK, CH), jnp.int32),
        [pltpu.VMEM((CH, OUT), jnp.float32)] * NB,
        pltpu.VMEM((RPS, OUT), jnp.float32),
        pltpu.VMEM((DROWS, 16), jnp.float32),
        pltpu.VMEM((5, 128), jnp.int32),
        pltpu.VMEM_SHARED((N, OUT), jnp.float32),
        pltpu.VMEM_SHARED((N, OUT), jnp.float32),
        pltpu.VMEM_SHARED((DROWS, 16), jnp.float32),
        pltpu.VMEM_SHARED((DROWS, 16), jnp.float32),
        [pltpu.SemaphoreType.DMA] * NB,
        [pltpu.SemaphoreType.DMA] * NB,
        pltpu.SemaphoreType.DMA,
    ],
)
def _sc_layer1(g_m, g_u, src_mu, dst_mu, src_um, dst_um,
               su_out, s1_out, dum_out, dmu_out,
               idx_s, idx_d, rows, zbuf, deg_v, idx_id, acc_u, acc_m,
               dsh_mu, dsh_um, gsems, ssems, wsem):
    cid = lax.axis_index("c")
    sid = lax.axis_index("s")
    wid = cid * NS + sid
    DPS = DROWS // NS

    _zero_fill(zbuf)
    _zero_acc(sid, zbuf, acc_u)
    _zero_acc(sid, zbuf, acc_m)
    _build_identity_idx(idx_id)
    _zero_deg(deg_v)
    pltpu.sync_copy(deg_v.at[pl.ds(sid * DPS, DPS)],
                    dsh_mu.at[pl.ds(sid * DPS, DPS)])
    pltpu.sync_copy(deg_v.at[pl.ds(sid * DPS, DPS)],
                    dsh_um.at[pl.ds(sid * DPS, DPS)])
    plsc.subcore_barrier()

    pltpu.sync_copy(src_mu.at[wid], idx_s)
    pltpu.sync_copy(dst_mu.at[wid], idx_d)
    _edge_pass(g_m, idx_s, idx_d, acc_u, rows, gsems, ssems, deg_v)
    _reduce_deg(deg_v, idx_id, dsh_mu)
    plsc.subcore_barrier()
    w_main, w_tail = _write_acc(cid, sid, acc_u, su_out, wsem)
    pltpu.sync_copy(dsh_mu.at[pl.ds(sid * DPS, DPS)],
                    dmu_out.at[cid, pl.ds(sid * DPS, DPS)])

    pltpu.sync_copy(src_um.at[wid], idx_s)
    pltpu.sync_copy(dst_um.at[wid], idx_d)
    _zero_deg(deg_v)
    _edge_pass(g_u, idx_s, idx_d, acc_m, rows, gsems, ssems, deg_v)
    _reduce_deg(deg_v, idx_id, dsh_um)
    plsc.subcore_barrier()
    pltpu.sync_copy(acc_m.at[pl.ds(sid * RPS, RPS)],
                    s1_out.at[cid, pl.ds(sid * RPS, RPS)])
    pltpu.sync_copy(dsh_um.at[pl.ds(sid * DPS, DPS)],
                    dum_out.at[cid, pl.ds(sid * DPS, DPS)])

    @pl.when(sid == NS - 1)
    def _():
        pltpu.sync_copy(acc_m.at[pl.ds(NS * RPS, 16)],
                        s1_out.at[cid, pl.ds(NS * RPS, 16)])

    _wait_acc(sid, w_main, w_tail)


@functools.partial(
    pl.kernel,
    out_type=jax.ShapeDtypeStruct((NC, N, OUT), jnp.float32),
    mesh=_mesh,
    compiler_params=_sc_params,
    scratch_types=[
        pltpu.VMEM((NCHUNK, CH), jnp.int32),
        pltpu.VMEM((NCHUNK, CH), jnp.int32),
        [pltpu.VMEM((CH, OUT), jnp.float32)] * NB,
        pltpu.VMEM((RPS, OUT), jnp.float32),
        pltpu.VMEM_SHARED((N, OUT), jnp.float32),
        [pltpu.SemaphoreType.DMA] * NB,
        [pltpu.SemaphoreType.DMA] * NB,
    ],
)
def _sc_segsum(table, src, dst, out, idx_s, idx_d, rows, zbuf, acc,
               gsems, ssems):
    cid = lax.axis_index("c")
    sid = lax.axis_index("s")
    wid = cid * NS + sid

    _zero_fill(zbuf)
    _zero_acc(sid, zbuf, acc)
    plsc.subcore_barrier()

    pltpu.sync_copy(src.at[wid], idx_s)
    pltpu.sync_copy(dst.at[wid], idx_d)
    _edge_pass(table, idx_s, idx_d, acc, rows, gsems, ssems, None)

    plsc.subcore_barrier()
    pltpu.sync_copy(acc.at[pl.ds(sid * RPS, RPS)],
                    out.at[cid, pl.ds(sid * RPS, RPS)])

    @pl.when(sid == NS - 1)
    def _():
        pltpu.sync_copy(acc.at[pl.ds(NS * RPS, 16)],
                        out.at[cid, pl.ds(NS * RPS, 16)])


BM = 1000


def _dense_body(x_ref, w1_ref, b1_ref, w2_ref, o_ref):
    h = jnp.dot(x_ref[...], w1_ref[0], preferred_element_type=jnp.float32)
    h = jnp.maximum(h + b1_ref[0], 0.0)
    o_ref[...] = jnp.dot(h, w2_ref[...], preferred_element_type=jnp.float32)


def _dense_proj(X, W1, B1, W2):
    nb = X.shape[0] // BM
    per = nb // 2
    return pl.pallas_call(
        _dense_body,
        grid=(nb,),
        in_specs=[
            pl.BlockSpec((BM, D), lambda i: (i, 0)),
            pl.BlockSpec((1, D, HID), lambda i: (i // per, 0, 0)),
            pl.BlockSpec((1, 1, HID), lambda i: (i // per, 0, 0)),
            pl.BlockSpec((HID, OUT), lambda i: (0, 0)),
        ],
        out_specs=pl.BlockSpec((BM, OUT), lambda i: (i, 0)),
        out_shape=jax.ShapeDtypeStruct((X.shape[0], OUT), jnp.float32),
    )(X, W1, B1, W2)


def _deg2_body(dmu_ref, dum_ref, du_ref, dm_ref):
    du_ref[...] = jnp.maximum(dmu_ref[0] + dmu_ref[1], 1.0)
    dm_ref[...] = jnp.maximum(dum_ref[0] + dum_ref[1], 1.0)


def _degsum2(dmu, dum):
    return pl.pallas_call(
        _deg2_body,
        grid=(1,),
        in_specs=[
            pl.BlockSpec((NC, DROWS, 16), lambda i: (0, 0, 0)),
            pl.BlockSpec((NC, DROWS, 16), lambda i: (0, 0, 0)),
        ],
        out_specs=[
            pl.BlockSpec((DROWS, 16), lambda i: (0, 0)),
            pl.BlockSpec((DROWS, 16), lambda i: (0, 0)),
        ],
        out_shape=[
            jax.ShapeDtypeStruct((DROWS, 16), jnp.float32),
            jax.ShapeDtypeStruct((DROWS, 16), jnp.float32),
        ],
    )(dmu, dum)


def _comb1_body(su_ref, d_ref, o_ref):
    o_ref[...] = (su_ref[0] + su_ref[1]) / d_ref[...]


def _combine1(su, deg):
    return pl.pallas_call(
        _comb1_body,
        grid=(N // BM,),
        in_specs=[
            pl.BlockSpec((2, BM, OUT), lambda i: (0, i, 0)),
            pl.BlockSpec((BM, 1), lambda i: (i, 0)),
        ],
        out_specs=pl.BlockSpec((BM, OUT), lambda i: (i, 0)),
        out_shape=jax.ShapeDtypeStruct((N, OUT), jnp.float32),
    )(su, deg)


def _final_body(g_ref, s1_ref, s2_ref, d_ref, b_ref, o_ref):
    num = (2.0 * ALPHA) * (s1_ref[0] + s1_ref[1]) + (s2_ref[0] + s2_ref[1])
    o_ref[...] = (ALPHA * ALPHA) * g_ref[...] + num / d_ref[...] + b_ref[...]


def _final(g_m, s1, s2, deg_m, b_out):
    return pl.pallas_call(
        _final_body,
        grid=(N // BM,),
        in_specs=[
            pl.BlockSpec((BM, OUT), lambda i: (i, 0)),
            pl.BlockSpec((2, BM, OUT), lambda i: (0, i, 0)),
            pl.BlockSpec((2, BM, OUT), lambda i: (0, i, 0)),
            pl.BlockSpec((BM, 1), lambda i: (i, 0)),
            pl.BlockSpec((1, OUT), lambda i: (0, 0)),
        ],
        out_specs=pl.BlockSpec((BM, OUT), lambda i: (i, 0)),
        out_shape=jax.ShapeDtypeStruct((N, OUT), jnp.float32),
    )(g_m, s1, s2, deg_m, b_out)


def kernel(x_movie, x_user, edge_index_um, edge_index_mu, W_in_movie,
           b_in_movie, W_in_user, b_in_user, W_out, b_out):
    X = jnp.concatenate([x_movie, x_user], axis=0)
    W1 = jnp.stack([W_in_movie, W_in_user])
    B1 = jnp.stack([b_in_movie, b_in_user]).reshape(2, 1, HID)
    G = _dense_proj(X, W1, B1, W_out)
    g_m, g_u = G[:N], G[N:]

    src_um = edge_index_um[0].astype(jnp.int32).reshape(NW, NCHUNK, CH)
    dst_um = edge_index_um[1].astype(jnp.int32).reshape(NW, NCHUNK, CH)
    src_mu = edge_index_mu[0].astype(jnp.int32).reshape(NW, NCHUNK, CH)
    dst_mu = edge_index_mu[1].astype(jnp.int32).reshape(NW, NCHUNK, CH)

    su, s1, dum, dmu = _sc_layer1(g_m, g_u, src_mu, dst_mu, src_um, dst_um)
    du, dm = _degsum2(dmu, dum)
    deg_u = du.reshape(DROWS * 16, 1)[:N]
    deg_m = dm.reshape(DROWS * 16, 1)[:N]
    t_u = _combine1(su, deg_u)
    s2 = _sc_segsum(t_u, src_um, dst_um)
    return _final(g_m, s1, s2, deg_m, b_out.reshape(1, OUT))


# speedup vs baseline: 1.3608x; 1.2442x over previous
import functools

import jax
import jax.numpy as jnp
from jax import lax
from jax.experimental import pallas as pl
from jax.experimental.pallas import tpu as pltpu
from jax.experimental.pallas import tpu_sc as plsc

N = 10000
E = 320000
D = 128
HID = 128
OUT = 32
ALPHA = 0.01

NC, NS = 2, 16
NW = NC * NS
CH = 128
NCHUNK = E // CH
MAXCH = NCHUNK // NW + 1
RPS = 624
NB = 4

_mesh = plsc.VectorSubcoreMesh(
    core_axis_name="c", subcore_axis_name="s", num_cores=NC, num_subcores=NS)

_sc_params = pltpu.CompilerParams(
    use_tc_tiling_on_sc=False, needs_layout_passes=False)


def _zero_fill(zbuf):
    z = jnp.zeros((16,), jnp.float32)

    def zb(j, carry):
        zbuf[j, pl.ds(0, 16)] = z
        zbuf[j, pl.ds(16, 16)] = z
        return carry

    lax.fori_loop(0, RPS, zb, 0)


DROWS = 640


def _zero_deg(deg_v):
    z = jnp.zeros((16,), jnp.float32)

    def zb(j, carry):
        deg_v[j, pl.ds(0, 16)] = z
        return carry

    lax.fori_loop(0, DROWS, zb, 0)


def _build_identity_idx(idx_id):
    base = lax.iota(jnp.int32, 16)
    for r in range(5):
        for m in range(8):
            idx_id[r, pl.ds(16 * m, 16)] = base + (128 * r + 16 * m)


def _reduce_deg(deg_v, idx_id, deg_sh):
    for r in range(5):
        pltpu.sync_copy(deg_v.at[pl.ds(128 * r, 128)],
                        deg_sh.at[idx_id.at[r]], add=True)


def _zero_acc(sid, zbuf, acc):
    pltpu.sync_copy(zbuf, acc.at[pl.ds(sid * RPS, RPS)])

    @pl.when(sid == NS - 1)
    def _():
        pltpu.sync_copy(zbuf.at[pl.ds(0, 16)], acc.at[pl.ds(NS * RPS, 16)])


def _write_acc(cid, sid, acc, out, sem):
    copies = [pltpu.make_async_copy(acc.at[pl.ds(sid * RPS, RPS)],
                                    out.at[cid, pl.ds(sid * RPS, RPS)], sem)]
    tail = pltpu.make_async_copy(acc.at[pl.ds(NS * RPS, 16)],
                                 out.at[cid, pl.ds(NS * RPS, 16)], sem)
    copies[0].start()

    @pl.when(sid == NS - 1)
    def _():
        tail.start()

    return copies[0], tail


def _wait_acc(sid, main, tail):
    main.wait()

    @pl.when(sid == NS - 1)
    def _():
        tail.wait()


def _edge_pass(table, idx_s, idx_d, nch, acc, rows, gsems, ssems, deg_v):
    ones = jnp.ones((16,), jnp.float32)

    def hist(j):
        if deg_v is None:
            return
        for k in range(CH // 16):
            v = idx_d[j, pl.ds(k * 16, 16)]
            plsc.addupdate_scatter(
                deg_v, [lax.shift_right_logical(v, 4),
                        lax.bitwise_and(v, 15)], ones)

    def step(j, b):
        bn = (b + NB - 1) % NB
        pltpu.make_async_copy(table.at[idx_s.at[j]], rows[b], gsems[b]).wait()
        pltpu.async_copy(rows[b], acc.at[idx_d.at[j]], ssems[b], add=True)
        hist(j)

        @pl.when(j + NB - 1 < nch)
        def _():
            @pl.when(j >= 1)
            def _():
                pltpu.make_async_copy(
                    rows[bn], acc.at[idx_d.at[j - 1]], ssems[bn]).wait()
            pltpu.async_copy(table.at[idx_s.at[j + NB - 1]], rows[bn],
                             gsems[bn])

    def eb(j, carry):
        for b in range(NB):
            @pl.when(j % NB == b)
            def _(b=b):
                step(j, b)
        return carry

    for k in range(NB - 1):
        pltpu.async_copy(table.at[idx_s.at[k]], rows[k], gsems[k])
    lax.fori_loop(0, nch, eb, 0)

    def drain(i, carry):
        j = nch - NB + i
        for b in range(NB):
            @pl.when(j % NB == b)
            def _(b=b):
                pltpu.make_async_copy(
                    rows[b], acc.at[idx_d.at[j]], ssems[b]).wait()
        return carry

    lax.fori_loop(0, NB, drain, 0)


@functools.partial(
    pl.kernel,
    out_type=(
        jax.ShapeDtypeStruct((NC, N, OUT), jnp.float32),
        jax.ShapeDtypeStruct((NC, N, OUT), jnp.float32),
        jax.ShapeDtypeStruct((NC, DROWS, 16), jnp.float32),
        jax.ShapeDtypeStruct((NC, DROWS, 16), jnp.float32),
    ),
    mesh=_mesh,
    compiler_params=_sc_params,
    scratch_types=[
        pltpu.VMEM((MAXCH, CH), jnp.int32),
        pltpu.VMEM((MAXCH, CH), jnp.int32),
        [pltpu.VMEM((CH, OUT), jnp.float32)] * NB,
        pltpu.VMEM((RPS, OUT), jnp.float32),
        pltpu.VMEM((DROWS, 16), jnp.float32),
        pltpu.VMEM((5, 128), jnp.int32),
        pltpu.VMEM_SHARED((N, OUT), jnp.float32),
        pltpu.VMEM_SHARED((N, OUT), jnp.float32),
        pltpu.VMEM_SHARED((DROWS, 16), jnp.float32),
        pltpu.VMEM_SHARED((DROWS, 16), jnp.float32),
        [pltpu.SemaphoreType.DMA] * NB,
        [pltpu.SemaphoreType.DMA] * NB,
        pltpu.SemaphoreType.DMA,
    ],
)
def _sc_layer1(g_m, g_u, e_mu, e_um,
               su_out, s1_out, dum_out, dmu_out,
               idx_s, idx_d, rows, zbuf, deg_v, idx_id, acc_u, acc_m,
               dsh_mu, dsh_um, gsems, ssems, wsem):
    cid = lax.axis_index("c")
    sid = lax.axis_index("s")
    wid = cid * NS + sid
    sw = (wid * NCHUNK) // NW
    nch = ((wid + 1) * NCHUNK) // NW - sw
    DPS = DROWS // NS

    _zero_fill(zbuf)
    _zero_acc(sid, zbuf, acc_u)
    _zero_acc(sid, zbuf, acc_m)
    _build_identity_idx(idx_id)
    _zero_deg(deg_v)
    pltpu.sync_copy(deg_v.at[pl.ds(sid * DPS, DPS)],
                    dsh_mu.at[pl.ds(sid * DPS, DPS)])
    pltpu.sync_copy(deg_v.at[pl.ds(sid * DPS, DPS)],
                    dsh_um.at[pl.ds(sid * DPS, DPS)])
    plsc.subcore_barrier()

    pltpu.sync_copy(e_mu.at[0, pl.ds(sw, MAXCH)], idx_s)
    pltpu.sync_copy(e_mu.at[1, pl.ds(sw, MAXCH)], idx_d)
    _edge_pass(g_m, idx_s, idx_d, nch, acc_u, rows, gsems, ssems, deg_v)
    _reduce_deg(deg_v, idx_id, dsh_mu)
    plsc.subcore_barrier()
    w_main, w_tail = _write_acc(cid, sid, acc_u, su_out, wsem)
    pltpu.sync_copy(dsh_mu.at[pl.ds(sid * DPS, DPS)],
                    dmu_out.at[cid, pl.ds(sid * DPS, DPS)])

    pltpu.sync_copy(e_um.at[0, pl.ds(sw, MAXCH)], idx_s)
    pltpu.sync_copy(e_um.at[1, pl.ds(sw, MAXCH)], idx_d)
    _zero_deg(deg_v)
    _edge_pass(g_u, idx_s, idx_d, nch, acc_m, rows, gsems, ssems, deg_v)
    _reduce_deg(deg_v, idx_id, dsh_um)
    plsc.subcore_barrier()
    pltpu.sync_copy(acc_m.at[pl.ds(sid * RPS, RPS)],
                    s1_out.at[cid, pl.ds(sid * RPS, RPS)])
    pltpu.sync_copy(dsh_um.at[pl.ds(sid * DPS, DPS)],
                    dum_out.at[cid, pl.ds(sid * DPS, DPS)])

    @pl.when(sid == NS - 1)
    def _():
        pltpu.sync_copy(acc_m.at[pl.ds(NS * RPS, 16)],
                        s1_out.at[cid, pl.ds(NS * RPS, 16)])

    _wait_acc(sid, w_main, w_tail)


@functools.partial(
    pl.kernel,
    out_type=jax.ShapeDtypeStruct((NC, N, OUT), jnp.float32),
    mesh=_mesh,
    compiler_params=_sc_params,
    scratch_types=[
        pltpu.VMEM((MAXCH, CH), jnp.int32),
        pltpu.VMEM((MAXCH, CH), jnp.int32),
        [pltpu.VMEM((CH, OUT), jnp.float32)] * NB,
        pltpu.VMEM((RPS, OUT), jnp.float32),
        pltpu.VMEM_SHARED((N, OUT), jnp.float32),
        [pltpu.SemaphoreType.DMA] * NB,
        [pltpu.SemaphoreType.DMA] * NB,
    ],
)
def _sc_segsum(table, edges, out, idx_s, idx_d, rows, zbuf, acc,
               gsems, ssems):
    cid = lax.axis_index("c")
    sid = lax.axis_index("s")
    wid = cid * NS + sid
    sw = (wid * NCHUNK) // NW
    nch = ((wid + 1) * NCHUNK) // NW - sw

    _zero_fill(zbuf)
    _zero_acc(sid, zbuf, acc)
    plsc.subcore_barrier()

    pltpu.sync_copy(edges.at[0, pl.ds(sw, MAXCH)], idx_s)
    pltpu.sync_copy(edges.at[1, pl.ds(sw, MAXCH)], idx_d)
    _edge_pass(table, idx_s, idx_d, nch, acc, rows, gsems, ssems, None)

    plsc.subcore_barrier()
    pltpu.sync_copy(acc.at[pl.ds(sid * RPS, RPS)],
                    out.at[cid, pl.ds(sid * RPS, RPS)])

    @pl.when(sid == NS - 1)
    def _():
        pltpu.sync_copy(acc.at[pl.ds(NS * RPS, 16)],
                        out.at[cid, pl.ds(NS * RPS, 16)])


BM = 1000


def _dense_body(x_ref, w1_ref, b1_ref, w2_ref, o_ref):
    h = jnp.dot(x_ref[...], w1_ref[...], preferred_element_type=jnp.float32)
    h = jnp.maximum(h + b1_ref[...], 0.0)
    o_ref[...] = jnp.dot(h, w2_ref[...], preferred_element_type=jnp.float32)


def _dense_proj(X, W1, B1, W2):
    nb = X.shape[0] // BM
    return pl.pallas_call(
        _dense_body,
        grid=(nb,),
        in_specs=[
            pl.BlockSpec((BM, D), lambda i: (i, 0)),
            pl.BlockSpec((D, HID), lambda i: (0, 0)),
            pl.BlockSpec((1, HID), lambda i: (0, 0)),
            pl.BlockSpec((HID, OUT), lambda i: (0, 0)),
        ],
        out_specs=pl.BlockSpec((BM, OUT), lambda i: (i, 0)),
        out_shape=jax.ShapeDtypeStruct((X.shape[0], OUT), jnp.float32),
    )(X, W1, B1, W2)


def _deg2_body(dmu_ref, dum_ref, du_ref, dm_ref):
    du_ref[...] = jnp.maximum(dmu_ref[0] + dmu_ref[1], 1.0)
    dm_ref[...] = jnp.maximum(dum_ref[0] + dum_ref[1], 1.0)


def _degsum2(dmu, dum):
    return pl.pallas_call(
        _deg2_body,
        grid=(1,),
        in_specs=[
            pl.BlockSpec((NC, DROWS, 16), lambda i: (0, 0, 0)),
            pl.BlockSpec((NC, DROWS, 16), lambda i: (0, 0, 0)),
        ],
        out_specs=[
            pl.BlockSpec((DROWS, 16), lambda i: (0, 0)),
            pl.BlockSpec((DROWS, 16), lambda i: (0, 0)),
        ],
        out_shape=[
            jax.ShapeDtypeStruct((DROWS, 16), jnp.float32),
            jax.ShapeDtypeStruct((DROWS, 16), jnp.float32),
        ],
    )(dmu, dum)


def _comb1_body(su_ref, d_ref, o_ref):
    o_ref[...] = (su_ref[0] + su_ref[1]) / d_ref[...]


def _combine1(su, deg):
    return pl.pallas_call(
        _comb1_body,
        grid=(N // BM,),
        in_specs=[
            pl.BlockSpec((2, BM, OUT), lambda i: (0, i, 0)),
            pl.BlockSpec((BM, 1), lambda i: (i, 0)),
        ],
        out_specs=pl.BlockSpec((BM, OUT), lambda i: (i, 0)),
        out_shape=jax.ShapeDtypeStruct((N, OUT), jnp.float32),
    )(su, deg)


def _final_body(g_ref, s1_ref, s2_ref, d_ref, b_ref, o_ref):
    num = (2.0 * ALPHA) * (s1_ref[0] + s1_ref[1]) + (s2_ref[0] + s2_ref[1])
    o_ref[...] = (ALPHA * ALPHA) * g_ref[...] + num / d_ref[...] + b_ref[...]


def _final(g_m, s1, s2, deg_m, b_out):
    return pl.pallas_call(
        _final_body,
        grid=(N // BM,),
        in_specs=[
            pl.BlockSpec((BM, OUT), lambda i: (i, 0)),
            pl.BlockSpec((2, BM, OUT), lambda i: (0, i, 0)),
            pl.BlockSpec((2, BM, OUT), lambda i: (0, i, 0)),
            pl.BlockSpec((BM, 1), lambda i: (i, 0)),
            pl.BlockSpec((1, OUT), lambda i: (0, 0)),
        ],
        out_specs=pl.BlockSpec((BM, OUT), lambda i: (i, 0)),
        out_shape=jax.ShapeDtypeStruct((N, OUT), jnp.float32),
    )(g_m, s1, s2, deg_m, b_out)


def kernel(x_movie, x_user, edge_index_um, edge_index_mu, W_in_movie,
           b_in_movie, W_in_user, b_in_user, W_out, b_out):
    g_m = _dense_proj(x_movie, W_in_movie, b_in_movie.reshape(1, HID), W_out)
    g_u = _dense_proj(x_user, W_in_user, b_in_user.reshape(1, HID), W_out)

    e_um = edge_index_um.astype(jnp.int32).reshape(2, NCHUNK, CH)
    e_mu = edge_index_mu.astype(jnp.int32).reshape(2, NCHUNK, CH)

    su, s1, dum, dmu = _sc_layer1(g_m, g_u, e_mu, e_um)
    du, dm = _degsum2(dmu, dum)
    deg_u = du.reshape(DROWS * 16, 1)[:N]
    deg_m = dm.reshape(DROWS * 16, 1)[:N]
    t_u = _combine1(su, deg_u)
    s2 = _sc_segsum(t_u, e_um)
    return _final(g_m, s1, s2, deg_m, b_out.reshape(1, OUT))


# speedup vs baseline: 1.4734x; 1.0827x over previous
import functools

import jax
import jax.numpy as jnp
from jax import lax
from jax.experimental import pallas as pl
from jax.experimental.pallas import tpu as pltpu
from jax.experimental.pallas import tpu_sc as plsc

N = 10000
E = 320000
D = 128
HID = 128
OUT = 32
ALPHA = 0.01

NC, NS = 2, 16
NW = NC * NS
CH = 128
NCHUNK = E // CH
MAXCH = NCHUNK // NW + 1
RPS = 624
SCQ = 208
NB = 4

_mesh = plsc.VectorSubcoreMesh(
    core_axis_name="c", subcore_axis_name="s", num_cores=NC, num_subcores=NS)

_sc_params = pltpu.CompilerParams(
    use_tc_tiling_on_sc=False, needs_layout_passes=False)


def _zero_fill(zbuf):
    z = jnp.zeros((16,), jnp.float32)

    def zb(j, carry):
        zbuf[j, pl.ds(0, 16)] = z
        zbuf[j, pl.ds(16, 16)] = z
        return carry

    lax.fori_loop(0, RPS, zb, 0)


DROWS = 640


def _zero_deg(deg_v):
    z = jnp.zeros((16,), jnp.float32)

    def zb(j, carry):
        deg_v[j, pl.ds(0, 16)] = z
        return carry

    lax.fori_loop(0, DROWS, zb, 0)


def _build_identity_idx(idx_id):
    base = lax.iota(jnp.int32, 16)
    for r in range(5):
        for m in range(8):
            idx_id[r, pl.ds(16 * m, 16)] = base + (128 * r + 16 * m)


def _reduce_deg(deg_v, idx_id, deg_sh):
    for r in range(5):
        pltpu.sync_copy(deg_v.at[pl.ds(128 * r, 128)],
                        deg_sh.at[idx_id.at[r]], add=True)


def _zero_acc(sid, zbuf, acc):
    pltpu.sync_copy(zbuf, acc.at[pl.ds(sid * RPS, RPS)])

    @pl.when(sid == NS - 1)
    def _():
        pltpu.sync_copy(zbuf.at[pl.ds(0, 16)], acc.at[pl.ds(NS * RPS, 16)])


def _write_acc(cid, sid, acc, out, sem):
    copies = [pltpu.make_async_copy(acc.at[pl.ds(sid * RPS, RPS)],
                                    out.at[cid, pl.ds(sid * RPS, RPS)], sem)]
    tail = pltpu.make_async_copy(acc.at[pl.ds(NS * RPS, 16)],
                                 out.at[cid, pl.ds(NS * RPS, 16)], sem)
    copies[0].start()

    @pl.when(sid == NS - 1)
    def _():
        tail.start()

    return copies[0], tail


def _wait_acc(sid, main, tail):
    main.wait()

    @pl.when(sid == NS - 1)
    def _():
        tail.wait()


def _edge_pass(table, idx_s, idx_d, nch, acc, rows, gsems, ssems, deg_v):
    ones = jnp.ones((16,), jnp.float32)

    def hist(j):
        if deg_v is None:
            return
        for k in range(CH // 16):
            v = idx_d[j, pl.ds(k * 16, 16)]
            plsc.addupdate_scatter(
                deg_v, [lax.shift_right_logical(v, 4),
                        lax.bitwise_and(v, 15)], ones)

    def step(j, b):
        bn = (b + NB - 1) % NB
        pltpu.make_async_copy(table.at[idx_s.at[j]], rows[b], gsems[b]).wait()
        pltpu.async_copy(rows[b], acc.at[idx_d.at[j]], ssems[b], add=True)
        hist(j)

        @pl.when(j + NB - 1 < nch)
        def _():
            @pl.when(j >= 1)
            def _():
                pltpu.make_async_copy(
                    rows[bn], acc.at[idx_d.at[j - 1]], ssems[bn]).wait()
            pltpu.async_copy(table.at[idx_s.at[j + NB - 1]], rows[bn],
                             gsems[bn])

    def eb(j, carry):
        for b in range(NB):
            @pl.when(j % NB == b)
            def _(b=b):
                step(j, b)
        return carry

    for k in range(NB - 1):
        pltpu.async_copy(table.at[idx_s.at[k]], rows[k], gsems[k])
    lax.fori_loop(0, nch, eb, 0)

    def drain(i, carry):
        j = nch - NB + i
        for b in range(NB):
            @pl.when(j % NB == b)
            def _(b=b):
                pltpu.make_async_copy(
                    rows[b], acc.at[idx_d.at[j]], ssems[b]).wait()
        return carry

    lax.fori_loop(0, NB, drain, 0)


@functools.partial(
    pl.kernel,
    out_type=(
        jax.ShapeDtypeStruct((NC, N, OUT), jnp.float32),
        jax.ShapeDtypeStruct((NC, N, OUT), jnp.float32),
        jax.ShapeDtypeStruct((NC, DROWS, 16), jnp.float32),
        jax.ShapeDtypeStruct((NC, DROWS, 16), jnp.float32),
    ),
    mesh=_mesh,
    compiler_params=_sc_params,
    scratch_types=[
        pltpu.VMEM((MAXCH, CH), jnp.int32),
        pltpu.VMEM((MAXCH, CH), jnp.int32),
        [pltpu.VMEM((CH, OUT), jnp.float32)] * NB,
        pltpu.VMEM((RPS, OUT), jnp.float32),
        pltpu.VMEM((DROWS, 16), jnp.float32),
        pltpu.VMEM((5, 128), jnp.int32),
        pltpu.VMEM_SHARED((N, OUT), jnp.float32),
        pltpu.VMEM_SHARED((N, OUT), jnp.float32),
        pltpu.VMEM_SHARED((DROWS, 16), jnp.float32),
        pltpu.VMEM_SHARED((DROWS, 16), jnp.float32),
        [pltpu.SemaphoreType.DMA] * NB,
        [pltpu.SemaphoreType.DMA] * NB,
        pltpu.SemaphoreType.DMA,
    ],
)
def _sc_layer1(g_m, g_u, e_mu, e_um,
               su_out, s1_out, dum_out, dmu_out,
               idx_s, idx_d, rows, zbuf, deg_v, idx_id, acc_u, acc_m,
               dsh_mu, dsh_um, gsems, ssems, wsem):
    cid = lax.axis_index("c")
    sid = lax.axis_index("s")
    wid = cid * NS + sid
    sw = (wid * NCHUNK) // NW
    nch = ((wid + 1) * NCHUNK) // NW - sw
    DPS = DROWS // NS

    _zero_fill(zbuf)
    _zero_acc(sid, zbuf, acc_u)
    _zero_acc(sid, zbuf, acc_m)
    _build_identity_idx(idx_id)
    _zero_deg(deg_v)
    pltpu.sync_copy(deg_v.at[pl.ds(sid * DPS, DPS)],
                    dsh_mu.at[pl.ds(sid * DPS, DPS)])
    pltpu.sync_copy(deg_v.at[pl.ds(sid * DPS, DPS)],
                    dsh_um.at[pl.ds(sid * DPS, DPS)])
    plsc.subcore_barrier()

    pltpu.sync_copy(e_mu.at[0, pl.ds(sw, MAXCH)], idx_s)
    pltpu.sync_copy(e_mu.at[1, pl.ds(sw, MAXCH)], idx_d)
    _edge_pass(g_m, idx_s, idx_d, nch, acc_u, rows, gsems, ssems, deg_v)
    _reduce_deg(deg_v, idx_id, dsh_mu)
    plsc.subcore_barrier()
    w_main, w_tail = _write_acc(cid, sid, acc_u, su_out, wsem)
    pltpu.sync_copy(dsh_mu.at[pl.ds(sid * DPS, DPS)],
                    dmu_out.at[cid, pl.ds(sid * DPS, DPS)])

    pltpu.sync_copy(e_um.at[0, pl.ds(sw, MAXCH)], idx_s)
    pltpu.sync_copy(e_um.at[1, pl.ds(sw, MAXCH)], idx_d)
    _zero_deg(deg_v)
    _edge_pass(g_u, idx_s, idx_d, nch, acc_m, rows, gsems, ssems, deg_v)
    _reduce_deg(deg_v, idx_id, dsh_um)
    plsc.subcore_barrier()
    pltpu.sync_copy(acc_m.at[pl.ds(sid * RPS, RPS)],
                    s1_out.at[cid, pl.ds(sid * RPS, RPS)])
    pltpu.sync_copy(dsh_um.at[pl.ds(sid * DPS, DPS)],
                    dum_out.at[cid, pl.ds(sid * DPS, DPS)])

    @pl.when(sid == NS - 1)
    def _():
        pltpu.sync_copy(acc_m.at[pl.ds(NS * RPS, 16)],
                        s1_out.at[cid, pl.ds(NS * RPS, 16)])

    _wait_acc(sid, w_main, w_tail)


@functools.partial(
    pl.kernel,
    out_type=jax.ShapeDtypeStruct((NC, N, OUT), jnp.float32),
    mesh=_mesh,
    compiler_params=_sc_params,
    scratch_types=[
        pltpu.VMEM((MAXCH, CH), jnp.int32),
        pltpu.VMEM((MAXCH, CH), jnp.int32),
        [pltpu.VMEM((CH, OUT), jnp.float32)] * NB,
        pltpu.VMEM((RPS, OUT), jnp.float32),
        pltpu.VMEM_SHARED((N, OUT), jnp.float32),
        [pltpu.SemaphoreType.DMA] * NB,
        [pltpu.SemaphoreType.DMA] * NB,
    ],
)
def _sc_segsum(table, edges, out, idx_s, idx_d, rows, zbuf, acc,
               gsems, ssems):
    cid = lax.axis_index("c")
    sid = lax.axis_index("s")
    wid = cid * NS + sid
    sw = (wid * NCHUNK) // NW
    nch = ((wid + 1) * NCHUNK) // NW - sw

    _zero_fill(zbuf)
    _zero_acc(sid, zbuf, acc)
    plsc.subcore_barrier()

    pltpu.sync_copy(edges.at[0, pl.ds(sw, MAXCH)], idx_s)
    pltpu.sync_copy(edges.at[1, pl.ds(sw, MAXCH)], idx_d)
    _edge_pass(table, idx_s, idx_d, nch, acc, rows, gsems, ssems, None)

    plsc.subcore_barrier()
    pltpu.sync_copy(acc.at[pl.ds(sid * RPS, RPS)],
                    out.at[cid, pl.ds(sid * RPS, RPS)])

    @pl.when(sid == NS - 1)
    def _():
        pltpu.sync_copy(acc.at[pl.ds(NS * RPS, 16)],
                        out.at[cid, pl.ds(NS * RPS, 16)])


@functools.partial(
    pl.kernel,
    out_type=jax.ShapeDtypeStruct((NC, N, OUT), jnp.float32),
    mesh=_mesh,
    compiler_params=_sc_params,
    scratch_types=[
        pltpu.VMEM((MAXCH, CH), jnp.int32),
        pltpu.VMEM((MAXCH, CH), jnp.int32),
        [pltpu.VMEM((CH, OUT), jnp.float32)] * NB,
        pltpu.VMEM((SCQ, OUT), jnp.float32),
        pltpu.VMEM((SCQ, OUT), jnp.float32),
        pltpu.VMEM((SCQ, OUT), jnp.float32),
        pltpu.VMEM((DROWS, 16), jnp.float32),
        pltpu.VMEM((DROWS, 16), jnp.float32),
        pltpu.VMEM_SHARED((N, OUT), jnp.float32),
        pltpu.VMEM_SHARED((N, OUT), jnp.float32),
        [pltpu.SemaphoreType.DMA] * NB,
        [pltpu.SemaphoreType.DMA] * NB,
    ],
)
def _sc_layer2(su, dmu, edges, out, idx_s, idx_d, rows, zbuf, su0_v, su1_v,
               deg_l, deg_l2, t_sh, acc, gsems, ssems):
    cid = lax.axis_index("c")
    sid = lax.axis_index("s")
    wid = cid * NS + sid
    sw = (wid * NCHUNK) // NW
    nch = ((wid + 1) * NCHUNK) // NW - sw
    base = sid * RPS

    z = jnp.zeros((16,), jnp.float32)

    def zb(j, carry):
        zbuf[j, pl.ds(0, 16)] = z
        zbuf[j, pl.ds(16, 16)] = z
        return carry

    lax.fori_loop(0, SCQ, zb, 0)
    for c in range(RPS // SCQ):
        pltpu.sync_copy(zbuf, acc.at[pl.ds(base + SCQ * c, SCQ)])

    @pl.when(sid == NS - 1)
    def _():
        pltpu.sync_copy(zbuf.at[pl.ds(0, 16)], acc.at[pl.ds(NS * RPS, 16)])

    pltpu.sync_copy(dmu.at[0], deg_l)
    pltpu.sync_copy(dmu.at[1], deg_l2)

    def dcomb(r, carry):
        deg_l[r, pl.ds(0, 16)] = jnp.maximum(
            deg_l[r, pl.ds(0, 16)] + deg_l2[r, pl.ds(0, 16)], 1.0)
        return carry

    lax.fori_loop(0, DROWS, dcomb, 0)

    for c in range(RPS // SCQ):
        off = base + SCQ * c
        pltpu.sync_copy(su.at[0, pl.ds(off, SCQ)], su0_v)
        pltpu.sync_copy(su.at[1, pl.ds(off, SCQ)], su1_v)

        def tgroup(g, carry, c=c):
            dinv = 1.0 / deg_l[sid * (RPS // 16) + c * (SCQ // 16) + g,
                               pl.ds(0, 16)]
            for j in range(16):
                i = g * 16 + j
                a0 = su0_v[i, pl.ds(0, 16)] + su1_v[i, pl.ds(0, 16)]
                a1 = su0_v[i, pl.ds(16, 16)] + su1_v[i, pl.ds(16, 16)]
                zbuf[i, pl.ds(0, 16)] = a0 * dinv[j]
                zbuf[i, pl.ds(16, 16)] = a1 * dinv[j]
            return carry

        lax.fori_loop(0, SCQ // 16, tgroup, 0)
        pltpu.sync_copy(zbuf, t_sh.at[pl.ds(off, SCQ)])

    @pl.when(sid == NS - 1)
    def _():
        pltpu.sync_copy(su.at[0, pl.ds(NS * RPS, 16)], su0_v.at[pl.ds(0, 16)])
        pltpu.sync_copy(su.at[1, pl.ds(NS * RPS, 16)], su1_v.at[pl.ds(0, 16)])
        dinv = 1.0 / deg_l[NS * (RPS // 16), pl.ds(0, 16)]
        for j in range(16):
            a0 = su0_v[j, pl.ds(0, 16)] + su1_v[j, pl.ds(0, 16)]
            a1 = su0_v[j, pl.ds(16, 16)] + su1_v[j, pl.ds(16, 16)]
            zbuf[j, pl.ds(0, 16)] = a0 * dinv[j]
            zbuf[j, pl.ds(16, 16)] = a1 * dinv[j]
        pltpu.sync_copy(zbuf.at[pl.ds(0, 16)], t_sh.at[pl.ds(NS * RPS, 16)])

    plsc.subcore_barrier()

    pltpu.sync_copy(edges.at[0, pl.ds(sw, MAXCH)], idx_s)
    pltpu.sync_copy(edges.at[1, pl.ds(sw, MAXCH)], idx_d)
    _edge_pass(t_sh, idx_s, idx_d, nch, acc, rows, gsems, ssems, None)

    plsc.subcore_barrier()
    pltpu.sync_copy(acc.at[pl.ds(sid * RPS, RPS)],
                    out.at[cid, pl.ds(sid * RPS, RPS)])

    @pl.when(sid == NS - 1)
    def _():
        pltpu.sync_copy(acc.at[pl.ds(NS * RPS, 16)],
                        out.at[cid, pl.ds(NS * RPS, 16)])


BM = 1000


def _dense_body(x_ref, w1_ref, b1_ref, w2_ref, o_ref):
    h = jnp.dot(x_ref[...], w1_ref[...], preferred_element_type=jnp.float32)
    h = jnp.maximum(h + b1_ref[...], 0.0)
    o_ref[...] = jnp.dot(h, w2_ref[...], preferred_element_type=jnp.float32)


def _dense_proj(X, W1, B1, W2):
    nb = X.shape[0] // BM
    return pl.pallas_call(
        _dense_body,
        grid=(nb,),
        in_specs=[
            pl.BlockSpec((BM, D), lambda i: (i, 0)),
            pl.BlockSpec((D, HID), lambda i: (0, 0)),
            pl.BlockSpec((1, HID), lambda i: (0, 0)),
            pl.BlockSpec((HID, OUT), lambda i: (0, 0)),
        ],
        out_specs=pl.BlockSpec((BM, OUT), lambda i: (i, 0)),
        out_shape=jax.ShapeDtypeStruct((X.shape[0], OUT), jnp.float32),
    )(X, W1, B1, W2)


def _deg2_body(dmu_ref, dum_ref, du_ref, dm_ref):
    du_ref[...] = jnp.maximum(dmu_ref[0] + dmu_ref[1], 1.0)
    dm_ref[...] = jnp.maximum(dum_ref[0] + dum_ref[1], 1.0)


def _degsum2(dmu, dum):
    return pl.pallas_call(
        _deg2_body,
        grid=(1,),
        in_specs=[
            pl.BlockSpec((NC, DROWS, 16), lambda i: (0, 0, 0)),
            pl.BlockSpec((NC, DROWS, 16), lambda i: (0, 0, 0)),
        ],
        out_specs=[
            pl.BlockSpec((DROWS, 16), lambda i: (0, 0)),
            pl.BlockSpec((DROWS, 16), lambda i: (0, 0)),
        ],
        out_shape=[
            jax.ShapeDtypeStruct((DROWS, 16), jnp.float32),
            jax.ShapeDtypeStruct((DROWS, 16), jnp.float32),
        ],
    )(dmu, dum)


def _comb1_body(su_ref, d_ref, o_ref):
    o_ref[...] = (su_ref[0] + su_ref[1]) / d_ref[...]


def _combine1(su, deg):
    return pl.pallas_call(
        _comb1_body,
        grid=(N // BM,),
        in_specs=[
            pl.BlockSpec((2, BM, OUT), lambda i: (0, i, 0)),
            pl.BlockSpec((BM, 1), lambda i: (i, 0)),
        ],
        out_specs=pl.BlockSpec((BM, OUT), lambda i: (i, 0)),
        out_shape=jax.ShapeDtypeStruct((N, OUT), jnp.float32),
    )(su, deg)


def _final_body(g_ref, s1_ref, s2_ref, d_ref, b_ref, o_ref):
    num = (2.0 * ALPHA) * (s1_ref[0] + s1_ref[1]) + (s2_ref[0] + s2_ref[1])
    o_ref[...] = (ALPHA * ALPHA) * g_ref[...] + num / d_ref[...] + b_ref[...]


def _final(g_m, s1, s2, deg_m, b_out):
    return pl.pallas_call(
        _final_body,
        grid=(N // BM,),
        in_specs=[
            pl.BlockSpec((BM, OUT), lambda i: (i, 0)),
            pl.BlockSpec((2, BM, OUT), lambda i: (0, i, 0)),
            pl.BlockSpec((2, BM, OUT), lambda i: (0, i, 0)),
            pl.BlockSpec((BM, 1), lambda i: (i, 0)),
            pl.BlockSpec((1, OUT), lambda i: (0, 0)),
        ],
        out_specs=pl.BlockSpec((BM, OUT), lambda i: (i, 0)),
        out_shape=jax.ShapeDtypeStruct((N, OUT), jnp.float32),
    )(g_m, s1, s2, deg_m, b_out)


def kernel(x_movie, x_user, edge_index_um, edge_index_mu, W_in_movie,
           b_in_movie, W_in_user, b_in_user, W_out, b_out):
    g_m = _dense_proj(x_movie, W_in_movie, b_in_movie.reshape(1, HID), W_out)
    g_u = _dense_proj(x_user, W_in_user, b_in_user.reshape(1, HID), W_out)

    e_um = edge_index_um.astype(jnp.int32).reshape(2, NCHUNK, CH)
    e_mu = edge_index_mu.astype(jnp.int32).reshape(2, NCHUNK, CH)

    su, s1, dum, dmu = _sc_layer1(g_m, g_u, e_mu, e_um)
    du, dm = _degsum2(dmu, dum)
    deg_m = dm.reshape(DROWS * 16, 1)[:N]
    s2 = _sc_layer2(su, dmu, e_um)
    return _final(g_m, s1, s2, deg_m, b_out.reshape(1, OUT))
